# baseline (speedup 1.0000x reference)
import functools

import jax
import jax.numpy as jnp
from jax import lax
from jax.experimental import pallas as pl
from jax.experimental.pallas import tpu as pltpu
from jax.experimental.pallas import tpu_sc as plsc

B = 16384
D = 16
_NC = 2
_NS = 16
_NW = _NC * _NS
_BPW = B // _NW
_CH = 128
_NCH = _BPW // _CH

_sc_mesh = plsc.VectorSubcoreMesh(core_axis_name="c", subcore_axis_name="s")


@functools.partial(
    pl.kernel,
    mesh=_sc_mesh,
    compiler_params=pltpu.CompilerParams(use_tc_tiling_on_sc=False),
    out_type=[
        jax.ShapeDtypeStruct((B, D), jnp.float32),
        jax.ShapeDtypeStruct((B, D), jnp.float32),
    ],
    scratch_types=[
        pltpu.VMEM((_NCH, _CH), jnp.int32),
        pltpu.VMEM((_NCH, _CH), jnp.int32),
        pltpu.VMEM((_BPW, D), jnp.float32),
        pltpu.VMEM((_BPW, D), jnp.float32),
        pltpu.SemaphoreType.DMA,
    ],
)
def _sc_gather(w_hbm, i0_hbm, i1_hbm, u_hbm, v_hbm, i0_v, i1_v, u_v, v_v, sem):
    wid = lax.axis_index("s") * _NC + lax.axis_index("c")
    row0 = wid * _NCH
    base = wid * _BPW
    pltpu.sync_copy(i0_hbm.at[pl.ds(row0, _NCH)], i0_v)
    pltpu.sync_copy(i1_hbm.at[pl.ds(row0, _NCH)], i1_v)
    copies = []
    for k in range(_NCH):
        copies.append(
            pltpu.async_copy(w_hbm.at[i0_v.at[k]], u_v.at[pl.ds(k * _CH, _CH)], sem))
        copies.append(
            pltpu.async_copy(w_hbm.at[i1_v.at[k]], v_v.at[pl.ds(k * _CH, _CH)], sem))
    for c in copies:
        c.wait()
    pltpu.sync_copy(u_v, u_hbm.at[pl.ds(base, _BPW)])
    pltpu.sync_copy(v_v, v_hbm.at[pl.ds(base, _BPW)])


def _tc_body(u_ref, v_ref, val_ref, out_ref, *, pairs):
    u = u_ref[...]
    v = v_ref[...]
    du = u - v
    z = 2.0 * jnp.sum(du * du, axis=1)
    su = jnp.sum(u * u, axis=1)
    sv = jnp.sum(v * v, axis=1)
    uu = 1.0 + z / ((1.0 - su) * (1.0 - sv))
    dist = jnp.log(uu + jnp.sqrt(uu * uu - 1.0))
    r = dist - val_ref[...]
    out_ref[0, 0] = jnp.sum(r * r) / pairs


def kernel(idx, values, w):
    n = w.shape[0]
    pairs = n * (n - 1) / 2.0
    idx32 = idx.astype(jnp.int32)
    i0 = idx32[:, 0].reshape(B // _CH, _CH)
    i1 = idx32[:, 1].reshape(B // _CH, _CH)
    u, v = _sc_gather(w, i0, i1)
    loss = pl.pallas_call(
        functools.partial(_tc_body, pairs=pairs),
        out_shape=jax.ShapeDtypeStruct((1, 1), jnp.float32),
        in_specs=[
            pl.BlockSpec(memory_space=pltpu.VMEM),
            pl.BlockSpec(memory_space=pltpu.VMEM),
            pl.BlockSpec(memory_space=pltpu.VMEM),
        ],
        out_specs=pl.BlockSpec(memory_space=pltpu.SMEM),
    )(u, v, values)
    return loss[0, 0]
